# Initial kernel scaffold; baseline (speedup 1.0000x reference)
import jax
import jax.numpy as jnp
from jax.experimental import pallas as pl


def kernel(x, unc, feature_prompt):
    raise NotImplementedError("write your pallas kernel here")


# speedup vs baseline: 2.1261x; 2.1261x over previous
import functools

import jax
import jax.numpy as jnp
from jax import lax
from jax.experimental import pallas as pl
from jax.experimental.pallas import tpu as pltpu
from jax.experimental.pallas import tpu_sc as plsc


def _sc_scatter_add(topk_idx, pvec_t, x2d):
    C, HW = x2d.shape
    K = topk_idx.shape[0]
    info = plsc.get_sparse_core_info()
    nw = info.num_cores * info.num_subcores
    cpw = C // nw

    mesh = plsc.VectorSubcoreMesh(core_axis_name="c", subcore_axis_name="s")

    @functools.partial(
        pl.kernel,
        mesh=mesh,
        out_type=jax.ShapeDtypeStruct((C, HW), jnp.float32),
        compiler_params=pltpu.CompilerParams(needs_layout_passes=False),
        scratch_types=[
            pltpu.VMEM((K,), jnp.int32),
            pltpu.VMEM((HW,), jnp.float32),
            pltpu.VMEM((K,), jnp.float32),
        ],
    )
    def k(idx_hbm, pv_hbm, x_hbm, out_hbm, idx_v, xbuf, pbuf):
        wid = lax.axis_index("s") * info.num_cores + lax.axis_index("c")
        pltpu.sync_copy(idx_hbm, idx_v)

        for kk in range(cpw):
            c = wid * cpw + kk
            pltpu.sync_copy(x_hbm.at[c], xbuf)
            pltpu.sync_copy(pv_hbm.at[c], pbuf)

            def body(j, _):
                sl = pl.ds(j * 16, 16)
                plsc.addupdate_scatter(xbuf, [idx_v[sl]], pbuf[sl])
                return None

            lax.fori_loop(0, K // 16, body, None)
            pltpu.sync_copy(xbuf, out_hbm.at[c])

    return k(topk_idx, pvec_t, x2d)


def kernel(x, unc, feature_prompt):
    pnum = feature_prompt.shape[0]
    _, C, H, W = x.shape
    flat = unc.reshape(-1)
    _, topk_idx = lax.top_k(flat, pnum)
    pvec_t = jnp.transpose(feature_prompt[:, :, 0, 0])
    x2d = x.reshape(C, H * W)
    out2d = _sc_scatter_add(topk_idx.astype(jnp.int32), pvec_t, x2d)
    return out2d.reshape(x.shape)
